# baseline (speedup 1.0000x reference)
import jax
import jax.numpy as jnp
from jax.experimental import pallas as pl

N = 10000
E = 320000
D = 128
H = 8
C = D // H
L = 3
B = 4096
G = 3


def _bn(h, g, b):
    mu = h.mean(axis=0, keepdims=True)
    var = ((h - mu) ** 2).mean(axis=0, keepdims=True)
    return (h - mu) / jnp.sqrt(var + 1e-5) * g + b


def kernel(x, W_lin, b_gat, att_src, att_dst, bn1_g, bn1_b, dW, db, bnd_g, bnd_b,
           uW, ub, bnu_g, bnu_b, hW1, hb1, hbn1_g, hbn1_b, hW2, hb2, hbn2_g, hbn2_b,
           gw, gb, edge_index, edges, groups):
    src, dst = edge_index[0], edge_index[1]
    h = x
    for l in range(L):
        xh = (h @ W_lin[l]).reshape(N, H, C)
        a_src = (xh * att_src[l][None]).sum(-1)
        a_dst = (xh * att_dst[l][None]).sum(-1)
        logits = jax.nn.leaky_relu(a_src[src] + a_dst[dst], 0.2)
        m = jax.ops.segment_max(logits, dst, num_segments=N)
        m = jnp.where(jnp.isfinite(m), m, 0.0)
        e = jnp.exp(logits - m[dst])
        denom = jax.ops.segment_sum(e, dst, num_segments=N)
        alpha = e / (denom[dst] + 1e-16)
        msg = (alpha[:, :, None] * xh[src]).reshape(E, H * C)
        out = jax.ops.segment_sum(msg, dst, num_segments=N)
        out = out + h + b_gat[l]
        h2 = jax.nn.elu(_bn(out, bn1_g[l], bn1_b[l]))
        bk = jax.nn.elu(_bn(h2 @ dW[l] + db[l], bnd_g[l], bnd_b[l]))
        bk = jax.nn.elu(_bn(bk @ uW[l] + ub[l], bnu_g[l], bnu_b[l]))
        h2 = h2 + bk
        h = h2 + h
    mu = h.mean(axis=0, keepdims=True)
    sd = jnp.std(h, axis=0, keepdims=True, ddof=1)
    h = (h - mu) / (sd + 1e-6)
    nrm = jnp.maximum(jnp.linalg.norm(h, axis=1, keepdims=True), 1e-12)
    h = h / nrm
    e_src, e_dst = edges[:, 0], edges[:, 1]
    z = jnp.concatenate([h[e_src], h[e_dst]], axis=-1)
    z = jax.nn.elu(_bn(z @ hW1 + hb1, hbn1_g, hbn1_b))
    z = jax.nn.elu(_bn(z @ hW2 + hb2, hbn2_g, hbn2_b))
    w = gw[groups]
    scores = jnp.einsum('bd,bdo->bo', z, w) + gb[groups]
    return scores.reshape(-1)


# speedup vs baseline: 10.5146x; 10.5146x over previous
import functools

import jax
import jax.numpy as jnp
from jax import lax
from jax.experimental import pallas as pl
from jax.experimental.pallas import tpu as pltpu
from jax.experimental.pallas import tpu_sc as plsc

N = 10000
E = 320000
D = 128
H = 8
C = D // H
L = 3
B = 4096
G = 3
BK = D // 4

NC = 2
NS = 16
NW = NC * NS

ROW = 16 + D
EPT = E // NW
CHUNK = 80
NCHUNK = EPT // CHUNK
DENBASE = N
DENROWS = 1250
ACCR = 11520
NPT = ACCR // NS

_NEG = -1.0e30


def _att_mat(attf):
    row = lax.broadcasted_iota(jnp.int32, (D, 8), 0)
    col = lax.broadcasted_iota(jnp.int32, (D, 8), 1)
    return jnp.where(row // C == col, jnp.broadcast_to(attf.T, (D, 8)), 0.0)


def _pre_body(h_ref, w_ref, as_ref, ad_ref, xh_ref, tatt_ref):
    h = h_ref[...]
    xh = jnp.dot(h, w_ref[...], preferred_element_type=jnp.float32)
    a_src = jnp.dot(xh, _att_mat(as_ref[...]),
                    preferred_element_type=jnp.float32)
    a_dst = jnp.dot(xh, _att_mat(ad_ref[...]),
                    preferred_element_type=jnp.float32)
    amax = jnp.max(a_src, axis=0, keepdims=True)
    m = jnp.maximum(amax + a_dst, 0.0)
    xh_ref[...] = xh
    tatt_ref[:, 0:8] = a_src
    tatt_ref[:, 8:16] = jnp.full((N, 8), _NEG, jnp.float32)
    tatt_ref[:, 16:24] = a_dst
    tatt_ref[:, 24:32] = jnp.zeros((N, 8), jnp.float32)
    tatt_ref[:, 32:40] = m
    tatt_ref[:, 40:48] = m
    tatt_ref[:, 48:D] = jnp.zeros((N, 80), jnp.float32)


def _tc_pre(h, w, att_s, att_d):
    return pl.pallas_call(
        _pre_body,
        out_shape=(
            jax.ShapeDtypeStruct((N, D), jnp.float32),
            jax.ShapeDtypeStruct((N, D), jnp.float32),
        ),
    )(h, w, att_s.reshape(1, D), att_d.reshape(1, D))


def _edge_body(xh_hbm, tatt_hbm, src_hbm, dst_hbm, zeros_hbm, omsg_hbm,
               sidx, didx, d8, g1, g2, g3, acc, sem1, sem2, sem3):
    cid = lax.axis_index("c")
    sid = lax.axis_index("s")
    wid = cid * NS + sid

    pltpu.sync_copy(zeros_hbm, acc.at[pl.ds(sid * NPT, NPT)])
    plsc.subcore_barrier()

    base_edge = wid * EPT

    def chunk_body(ci, carry):
        base = base_edge + ci * CHUNK
        pltpu.sync_copy(src_hbm.at[pl.ds(base, CHUNK)], sidx)
        pltpu.sync_copy(dst_hbm.at[pl.ds(base, CHUNK)], didx)
        cp1 = pltpu.async_copy(xh_hbm.at[sidx], g1, sem1)
        cp2 = pltpu.async_copy(tatt_hbm.at[didx], g2, sem2)
        cp3 = pltpu.async_copy(tatt_hbm.at[sidx], g3, sem3)
        cp1.wait()
        cp2.wait()
        cp3.wait()

        def group_body(jj, c2):
            dvec = didx[pl.ds(jj * 16, 16)]
            d8[pl.ds(jj * 16, 16)] = DENBASE + (dvec >> 3)
            for jo in range(16):
                j = jj * 16 + jo
                va = g3[j, 0:16]
                vb = g2[j, 16:32]
                vm = g2[j, 32:48]
                t = va + vb
                lr = jnp.maximum(t, 0.2 * t)
                e = jnp.exp(lr - vm)
                doff = 48 + ((dvec[jo] & 7) << 3)
                g2[j, pl.ds(doff, 16)] = e
                for hh in range(H):
                    eh = e[hh]
                    sl = pl.ds(C * hh, C)
                    g1[j, sl] = g1[j, sl] * eh
            return c2

        lax.fori_loop(0, CHUNK // 16, group_body, 0)
        pltpu.sync_copy(g1, acc.at[didx], add=True)
        pltpu.sync_copy(g2, acc.at[d8], add=True)
        return carry

    lax.fori_loop(0, NCHUNK, chunk_body, 0)

    plsc.subcore_barrier()
    pltpu.sync_copy(acc.at[pl.ds(sid * NPT, NPT)],
                    omsg_hbm.at[cid, pl.ds(sid * NPT, NPT)])


_edge_kernel = functools.partial(
    pl.kernel,
    _edge_body,
    out_type=jax.ShapeDtypeStruct((NC, ACCR, D), jnp.float32),
    mesh=plsc.VectorSubcoreMesh(core_axis_name="c", subcore_axis_name="s"),
    scratch_types=[
        pltpu.VMEM((CHUNK,), jnp.int32),
        pltpu.VMEM((CHUNK,), jnp.int32),
        pltpu.VMEM((CHUNK,), jnp.int32),
        pltpu.VMEM((CHUNK, D), jnp.float32),
        pltpu.VMEM((CHUNK, D), jnp.float32),
        pltpu.VMEM((CHUNK, D), jnp.float32),
        pltpu.VMEM_SHARED((ACCR, D), jnp.float32),
        pltpu.SemaphoreType.DMA,
        pltpu.SemaphoreType.DMA,
        pltpu.SemaphoreType.DMA,
    ],
    compiler_params=pltpu.CompilerParams(needs_layout_passes=False),
)()


def _bn_tc(h, g, b):
    mu = h.mean(axis=0, keepdims=True)
    var = ((h - mu) ** 2).mean(axis=0, keepdims=True)
    return (h - mu) * jax.lax.rsqrt(var + 1e-5) * g + b


def _elu(x):
    return jnp.where(x > 0, x, jnp.exp(jnp.minimum(x, 0.0)) - 1.0)


def _post_body(a0_ref, a1_ref, h_ref, bg_ref, g1_ref, b1_ref, dw_ref,
               db_ref, gd_ref, bd_ref, uw_ref, ub_ref, gu_ref, bu_ref, out_ref):
    msg = a0_ref[0:N] + a1_ref[0:N]
    den = a0_ref[DENBASE:DENBASE + DENROWS] + a1_ref[DENBASE:DENBASE + DENROWS]
    inv = 1.0 / (den + 1e-16)
    ih = lax.broadcasted_iota(jnp.int32, (8, D), 0)
    il = lax.broadcasted_iota(jnp.int32, (8, D), 1) // C
    ek = (ih == il).astype(jnp.float32)
    parts = [jnp.dot(inv[:, 48 + 8 * j:56 + 8 * j], ek,
                     preferred_element_type=jnp.float32) for j in range(8)]
    expand = jnp.stack(parts, axis=1).reshape(N, D)
    out = msg * expand
    h = h_ref[...]
    out = out + h + bg_ref[...]
    h2 = _elu(_bn_tc(out, g1_ref[...], b1_ref[...]))
    bk = _elu(_bn_tc(jnp.dot(h2, dw_ref[...], preferred_element_type=jnp.float32)
                     + db_ref[...], gd_ref[...], bd_ref[...]))
    bk = _elu(_bn_tc(jnp.dot(bk, uw_ref[...], preferred_element_type=jnp.float32)
                     + ub_ref[...], gu_ref[...], bu_ref[...]))
    out_ref[...] = h2 + bk + h


def _tc_post(a0, a1, h, bg, g1, b1, dw, db, gd, bd, uw, ub, gu, bu):
    return pl.pallas_call(
        _post_body,
        out_shape=jax.ShapeDtypeStruct((N, D), jnp.float32),
    )(a0, a1, h, bg.reshape(1, D), g1.reshape(1, D), b1.reshape(1, D),
      dw, db.reshape(1, BK), gd.reshape(1, BK), bd.reshape(1, BK),
      uw, ub.reshape(1, D), gu.reshape(1, D), bu.reshape(1, D))


def _norm_body(h_ref, out_ref):
    h = h_ref[...]
    mu = h.mean(axis=0, keepdims=True)
    ss = ((h - mu) ** 2).sum(axis=0, keepdims=True)
    sd = jnp.sqrt(ss / (N - 1))
    h = (h - mu) / (sd + 1e-6)
    nrm = jnp.maximum(jnp.sqrt((h * h).sum(axis=1, keepdims=True)), 1e-12)
    out_ref[...] = h / nrm


def _tc_norm(h):
    return pl.pallas_call(
        _norm_body,
        out_shape=jax.ShapeDtypeStruct((N, D), jnp.float32),
    )(h)


BPT = B // NW


def _hgather_body(h_hbm, es_hbm, ed_hbm, z_hbm, sidx, didx, gs, gd, sem1, sem2):
    cid = lax.axis_index("c")
    sid = lax.axis_index("s")
    wid = cid * NS + sid
    base = wid * BPT
    pltpu.sync_copy(es_hbm.at[pl.ds(base, BPT)], sidx)
    pltpu.sync_copy(ed_hbm.at[pl.ds(base, BPT)], didx)
    cp1 = pltpu.async_copy(h_hbm.at[sidx], gs, sem1)
    cp2 = pltpu.async_copy(h_hbm.at[didx], gd, sem2)
    cp1.wait()
    cp2.wait()
    pltpu.sync_copy(gs, z_hbm.at[0, pl.ds(base, BPT)])
    pltpu.sync_copy(gd, z_hbm.at[1, pl.ds(base, BPT)])


_hgather_kernel = functools.partial(
    pl.kernel,
    _hgather_body,
    out_type=jax.ShapeDtypeStruct((2, B, D), jnp.float32),
    mesh=plsc.VectorSubcoreMesh(core_axis_name="c", subcore_axis_name="s"),
    scratch_types=[
        pltpu.VMEM((BPT,), jnp.int32),
        pltpu.VMEM((BPT,), jnp.int32),
        pltpu.VMEM((BPT, D), jnp.float32),
        pltpu.VMEM((BPT, D), jnp.float32),
        pltpu.SemaphoreType.DMA,
        pltpu.SemaphoreType.DMA,
    ],
    compiler_params=pltpu.CompilerParams(needs_layout_passes=False),
)()


def _head_body(z0_ref, z1_ref, grp_ref, w1a_ref, w1b_ref, b1_ref, g1_ref,
               bb1_ref, w2_ref, b2_ref, g2_ref, bb2_ref, gw_ref, gb_ref,
               out_ref):
    z = (jnp.dot(z0_ref[...], w1a_ref[...], preferred_element_type=jnp.float32)
         + jnp.dot(z1_ref[...], w1b_ref[...], preferred_element_type=jnp.float32)
         + b1_ref[...])
    z = _elu(_bn_tc(z, g1_ref[...], bb1_ref[...]))
    z = jnp.dot(z, w2_ref[...], preferred_element_type=jnp.float32) + b2_ref[...]
    z = _elu(_bn_tc(z, g2_ref[...], bb2_ref[...]))
    s3 = jnp.dot(z, gw_ref[...], preferred_element_type=jnp.float32)
    grp = grp_ref[...]
    sel = (grp == lax.broadcasted_iota(jnp.int32, (B, G), 1)).astype(jnp.float32)
    out_ref[...] = (s3 * sel).sum(axis=1, keepdims=True) \
        + jnp.dot(sel, gb_ref[...], preferred_element_type=jnp.float32)


def _tc_head(z0, z1, grp, hW1, hb1, hbn1_g, hbn1_b, hW2, hb2, hbn2_g, hbn2_b,
             gw, gb):
    return pl.pallas_call(
        _head_body,
        out_shape=jax.ShapeDtypeStruct((B, 1), jnp.float32),
    )(z0, z1, grp.reshape(B, 1), hW1[:D], hW1[D:], hb1.reshape(1, D),
      hbn1_g.reshape(1, D), hbn1_b.reshape(1, D), hW2, hb2.reshape(1, D // 2),
      hbn2_g.reshape(1, D // 2), hbn2_b.reshape(1, D // 2),
      gw.reshape(G, D // 2).T, gb.reshape(G, 1))


def kernel(x, W_lin, b_gat, att_src, att_dst, bn1_g, bn1_b, dW, db, bnd_g, bnd_b,
           uW, ub, bnu_g, bnu_b, hW1, hb1, hbn1_g, hbn1_b, hW2, hb2, hbn2_g, hbn2_b,
           gw, gb, edge_index, edges, groups):
    src = edge_index[0]
    dst = edge_index[1]
    zeros = jnp.zeros((NPT, D), jnp.float32)
    h = x
    for l in range(L):
        xh, tatt = _tc_pre(h, W_lin[l], att_src[l], att_dst[l])
        omsg = _edge_kernel(xh, tatt, src, dst, zeros)
        h = _tc_post(omsg[0], omsg[1], h, b_gat[l], bn1_g[l], bn1_b[l],
                     dW[l], db[l], bnd_g[l], bnd_b[l],
                     uW[l], ub[l], bnu_g[l], bnu_b[l])
    h = _tc_norm(h)
    z0 = _hgather_kernel(h, edges[:, 0], edges[:, 1])
    scores = _tc_head(z0[0], z0[1], groups, hW1, hb1, hbn1_g, hbn1_b,
                      hW2, hb2, hbn2_g, hbn2_b, gw, gb)
    return scores.reshape(-1)
